# baseline (speedup 1.0000x reference)
import functools

import jax
import jax.numpy as jnp
from jax import lax
from jax.experimental import pallas as pl
from jax.experimental.pallas import tpu as pltpu
from jax.experimental.pallas import tpu_sc as plsc

D_MODEL = 64
IDX_ROW = 128
RING = 8
WIN = 6


def _embed_sc(x_rows, table, n_rows):
    info = plsc.get_sparse_core_info()
    nc, ns = info.num_cores, info.num_subcores
    nw = nc * ns
    units_per_w = n_rows // nw
    b = n_rows * IDX_ROW

    mesh = plsc.VectorSubcoreMesh(core_axis_name="c", subcore_axis_name="s")

    @functools.partial(
        pl.kernel,
        mesh=mesh,
        out_type=jax.ShapeDtypeStruct((b, D_MODEL), jnp.float32),
        scratch_types=[
            pltpu.VMEM((RING, IDX_ROW), jnp.int32),
            pltpu.VMEM((RING, IDX_ROW, D_MODEL), jnp.float32),
            pltpu.SemaphoreType.DMA((RING,)),
            pltpu.SemaphoreType.DMA((RING,)),
            pltpu.SemaphoreType.DMA((RING,)),
        ],
        compiler_params=pltpu.CompilerParams(use_tc_tiling_on_sc=False),
    )
    def k(x_hbm, table_hbm, out_hbm, idx_v, rows_v, isem, gsem, osem):
        wid = lax.axis_index("s") * nc + lax.axis_index("c")
        wrow = wid * units_per_w

        def fire_idx(u, slot):
            pltpu.async_copy(x_hbm.at[wrow + u], idx_v.at[slot], isem.at[slot])

        def wait_idx(slot):
            pltpu.make_async_copy(
                x_hbm.at[0], idx_v.at[0], isem.at[slot]
            ).wait()

        def fire_gather(slot):
            pltpu.async_copy(
                table_hbm.at[idx_v.at[slot]], rows_v.at[slot], gsem.at[slot]
            )

        def wait_gather(slot):
            pltpu.make_async_copy(
                table_hbm.at[idx_v.at[0]], rows_v.at[0], gsem.at[slot]
            ).wait()

        def fire_store(v, slot):
            pltpu.async_copy(
                rows_v.at[slot],
                out_hbm.at[pl.ds((wrow + v) * IDX_ROW, IDX_ROW)],
                osem.at[slot],
            )

        def drain_store(slot):
            pltpu.make_async_copy(
                rows_v.at[0], out_hbm.at[pl.ds(0, IDX_ROW)], osem.at[slot]
            ).wait()

        for j in range(RING):
            fire_idx(j, j)

        def step(u, carry):
            slot = u % RING

            @pl.when(u >= RING)
            def _():
                drain_store(slot)

            wait_idx(slot)
            fire_gather(slot)

            @pl.when(u >= WIN)
            def _():
                v = u - WIN
                sv = v % RING
                wait_gather(sv)
                fire_store(v, sv)

                @pl.when(v + RING < units_per_w)
                def _():
                    fire_idx(v + RING, sv)

            return carry

        lax.fori_loop(0, units_per_w, step, 0)

        for v in range(units_per_w - WIN, units_per_w):
            sv = v % RING
            wait_gather(sv)
            fire_store(v, sv)
        for v in range(units_per_w - RING, units_per_w):
            drain_store(v % RING)

    return k(x_rows, table)


def kernel(x, table):
    orig_shape = x.shape
    xf = x.reshape(-1, IDX_ROW).astype(jnp.int32)
    out = _embed_sc(xf, table, xf.shape[0])
    return out.reshape(*orig_shape, D_MODEL)
